# baseline (speedup 1.0000x reference)
import functools
import jax
import jax.numpy as jnp
from jax.experimental import pallas as pl

B, P, A, E, C = 128, 8, 12, 512, 2
S = 1 + A
ROWS = P * S
EC = E * C
DELTA = 1e-20


def _flow_body(xb_ref, xf_ref, wfull_ref, w2_ref, bin_ref, b24_ref,
               rew_ref, pif_ref, iflow_ref, out_ref):
    xb = xb_ref[0]
    prod = xb * wfull_ref[...]
    par = jax.lax.broadcasted_iota(jnp.int32, (1, EC), 1) % 2
    sgn = (1 - 2 * par).astype(jnp.float32)
    tot = jnp.sum(prod, axis=-1, keepdims=True)
    dif = jnp.sum(prod * sgn, axis=-1, keepdims=True)
    zin = jnp.concatenate([(tot + dif) * 0.5, (tot - dif) * 0.5], axis=-1)
    sp_in = jax.nn.softplus(zin + bin_ref[...])
    slot = jax.lax.broadcasted_iota(jnp.int32, (ROWS, 1), 0) % S
    sp_in = jnp.where(slot == 0, 0.0, sp_in)
    gi = jax.lax.broadcasted_iota(jnp.int32, (P, ROWS), 0)
    gr = jax.lax.broadcasted_iota(jnp.int32, (P, ROWS), 1)
    g = (gi == gr // S).astype(jnp.float32)
    f_in = jnp.dot(g, sp_in, preferred_element_type=jnp.float32)

    zf = jnp.dot(xf_ref[0], w2_ref[...],
                 preferred_element_type=jnp.float32)
    sp_f = jax.nn.softplus(zf + b24_ref[...])
    par24 = jax.lax.broadcasted_iota(jnp.int32, (1, A * C), 1) % 2
    sgn24 = (1 - 2 * par24).astype(jnp.float32)
    tot_f = jnp.sum(sp_f, axis=-1, keepdims=True)
    dif_f = jnp.sum(sp_f * sgn24, axis=-1, keepdims=True)
    f_out = jnp.concatenate([(tot_f + dif_f) * 0.5,
                             (tot_f - dif_f) * 0.5], axis=-1)

    rew = rew_ref[...]
    logterm = jnp.log(DELTA + f_out) - jnp.log(DELTA + f_out + rew)
    li = jax.lax.broadcasted_iota(jnp.int32, (P, P), 0)
    lj = jax.lax.broadcasted_iota(jnp.int32, (P, P), 1)
    ltri = (lj < li).astype(jnp.float32)
    p_ex = jnp.dot(ltri, logterm, preferred_element_type=jnp.float32)

    f_init = pif_ref[...] * jnp.exp(iflow_ref[...])

    out_ref[...] = jnp.concatenate(
        [f_in, f_out, rew, f_init, p_ex, rew], axis=-1)


def kernel(forward_edges, backward_edges, path_init_flow, paths_reward,
           W, b, initial_flow):
    f32 = jnp.float32
    xb = backward_edges.reshape(B, ROWS, EC)
    xf = forward_edges[:, :, 0, :, :].reshape(B, P, EC)

    wslot = jnp.zeros((S, E, C), f32).at[1:].set(jnp.transpose(W, (2, 1, 0)))
    wfull = jnp.tile(wslot.reshape(S, EC), (P, 1))
    w2 = jnp.einsum('cea,cd->edac', W,
                    jnp.eye(C, dtype=f32)).reshape(EC, A * C)
    bslot = jnp.zeros((S, C), f32).at[1:].set(b.T)
    bias_in = jnp.tile(bslot, (P, 1))
    bias24 = b.T.reshape(1, A * C)
    rew2d = paths_reward.reshape(B * P, C)
    pif2d = path_init_flow.reshape(B * P, C)
    iflow2d = initial_flow.reshape(1, C)

    out = pl.pallas_call(
        _flow_body,
        grid=(B,),
        in_specs=[
            pl.BlockSpec((1, ROWS, EC), lambda i: (i, 0, 0)),
            pl.BlockSpec((1, P, EC), lambda i: (i, 0, 0)),
            pl.BlockSpec((ROWS, EC), lambda i: (0, 0)),
            pl.BlockSpec((EC, A * C), lambda i: (0, 0)),
            pl.BlockSpec((ROWS, C), lambda i: (0, 0)),
            pl.BlockSpec((1, A * C), lambda i: (0, 0)),
            pl.BlockSpec((P, C), lambda i: (i, 0)),
            pl.BlockSpec((P, C), lambda i: (i, 0)),
            pl.BlockSpec((1, C), lambda i: (0, 0)),
        ],
        out_specs=pl.BlockSpec((P, 12), lambda i: (i, 0)),
        out_shape=jax.ShapeDtypeStruct((B * P, 12), f32),
    )(xb, xf, wfull, w2, bias_in, bias24, rew2d, pif2d, iflow2d)

    return out.reshape(B, P, 6, C).swapaxes(-1, -2)
